# baseline (speedup 1.0000x reference)
import functools
import math

import jax
import jax.numpy as jnp
from jax import lax
from jax.experimental import pallas as pl
from jax.experimental.pallas import tpu as pltpu
from jax.experimental.pallas import tpu_sc as plsc

_VOCAB = 1000000
_DIM = 64
_B = 4096
_L = 200

_NC = 2
_NS = 16
_NW = _NC * _NS

_BSLAB = _B // _NW
_NBUF = 4
_DT = _DIM // 8


def _pos_table():
    def body(o_ref):
        i = lax.broadcasted_iota(jnp.int32, (_L, _DIM), 0).astype(jnp.float32)
        d = lax.broadcasted_iota(jnp.int32, (_L, _DIM), 1)
        d_even = ((d // 2) * 2).astype(jnp.float32)
        angle = i * jnp.exp(d_even * (-math.log(10000.0) / _DIM))
        o_ref[...] = jnp.where(d % 2 == 0, jnp.sin(angle), jnp.cos(angle))

    return pl.pallas_call(
        body, out_shape=jax.ShapeDtypeStruct((_L, _DIM), jnp.float32)
    )()


@functools.partial(
    pl.kernel,
    mesh=plsc.VectorSubcoreMesh(core_axis_name="c", subcore_axis_name="s"),
    out_type=jax.ShapeDtypeStruct((_L, _DT, _NW, 8, _BSLAB), jnp.float32),
    scratch_types=[
        pltpu.VMEM((_L, _BSLAB), jnp.int32),
        pltpu.VMEM((_L, _DIM), jnp.float32),
        pltpu.VMEM((_NBUF, _BSLAB, _DIM), jnp.float32),
        pltpu.VMEM((_NBUF, _DT, 8, _BSLAB), jnp.float32),
        pltpu.SemaphoreType.DMA,
        pltpu.SemaphoreType.DMA,
    ],
    compiler_params=pltpu.CompilerParams(
        use_tc_tiling_on_sc=False, needs_layout_passes=False
    ),
)
def _sc_lookup(table_hbm, xt_hbm, pos_hbm, out_hbm, idx_all, pos_v, rows_v,
               out_v, gsem, osem):
    wid = lax.axis_index("s") * _NC + lax.axis_index("c")
    b0 = wid * _BSLAB

    pltpu.sync_copy(pos_hbm, pos_v)
    pltpu.sync_copy(xt_hbm.at[:, pl.ds(b0, _BSLAB)], idx_all)

    iota16 = lax.iota(jnp.int32, 16)
    row_idx = [iota16 + 16 * k for k in range(_BSLAB // 16)]

    def fire_gather(l, b):
        pltpu.async_copy(table_hbm.at[idx_all.at[l]], rows_v.at[b], gsem)

    def wait_gather(b):
        pltpu.make_async_copy(
            table_hbm.at[idx_all.at[0]], rows_v.at[b], gsem
        ).wait()

    def fire_out(l, b):
        pltpu.async_copy(out_v.at[b], out_hbm.at[l, :, wid], osem)

    def wait_out():
        pltpu.make_async_copy(out_v.at[0], out_hbm.at[0, :, 0], osem).wait()

    def transpose_add(l, b):
        l_vec = jnp.full((16,), l, jnp.int32)

        @plsc.parallel_loop(0, _DIM, 1, unroll=2)
        def _(d):
            g = d // 8
            ds_ = d % 8
            col = jnp.full((16,), d, jnp.int32)
            p = plsc.load_gather(pos_v, [l_vec, col])
            for k in range(_BSLAB // 16):
                v = plsc.load_gather(rows_v.at[b], [row_idx[k], col])
                out_v[b, g, ds_, pl.ds(k * 16, 16)] = v + p

    def step(l, b, fire_ahead):
        if fire_ahead:
            wait_out()
            fire_gather(l + 2, (b + 2) % _NBUF)
        wait_gather(b)
        transpose_add(l, b)
        fire_out(l, b)

    pltpu.async_copy(out_v.at[2], out_hbm.at[0, :, wid], osem)
    pltpu.async_copy(out_v.at[3], out_hbm.at[1, :, wid], osem)
    fire_gather(0, 0)
    fire_gather(1, 1)

    for l in range(4):
        step(l, l % _NBUF, True)

    def group(g, carry):
        l0 = 4 + g * _NBUF
        for k in range(_NBUF):
            step(l0 + k, k, True)
        return carry

    lax.fori_loop(0, (_L - 8) // _NBUF, group, 0)

    for l in range(_L - 4, _L):
        step(l, l % _NBUF, l + 2 < _L)

    for _ in range(4):
        wait_out()


def kernel(x, table):
    pos = _pos_table()
    xt = jnp.transpose(x)
    raw = _sc_lookup(table, xt, pos)
    return jnp.transpose(raw, (2, 4, 0, 1, 3)).reshape(_B, _L, _DIM)


# speedup vs baseline: 1.5892x; 1.5892x over previous
import functools
import math

import jax
import jax.numpy as jnp
from jax import lax
from jax.experimental import pallas as pl
from jax.experimental.pallas import tpu as pltpu
from jax.experimental.pallas import tpu_sc as plsc

_VOCAB = 1000000
_DIM = 64
_B = 4096
_L = 200

_NC = 2
_NS = 16
_NW = _NC * _NS

_BSLAB = _B // _NW
_NBUF = 4
_DT = _DIM // 8


def _pos_table():
    def body(o_ref):
        i = lax.broadcasted_iota(jnp.int32, (_L, _DIM), 0).astype(jnp.float32)
        d = lax.broadcasted_iota(jnp.int32, (_L, _DIM), 1)
        d_even = ((d // 2) * 2).astype(jnp.float32)
        angle = i * jnp.exp(d_even * (-math.log(10000.0) / _DIM))
        o_ref[...] = jnp.where(d % 2 == 0, jnp.sin(angle), jnp.cos(angle))

    return pl.pallas_call(
        body, out_shape=jax.ShapeDtypeStruct((_L, _DIM), jnp.float32)
    )()


@functools.partial(
    pl.kernel,
    mesh=plsc.VectorSubcoreMesh(core_axis_name="c", subcore_axis_name="s"),
    out_type=jax.ShapeDtypeStruct((_L, _DT, _NW, 8 * _BSLAB), jnp.float32),
    scratch_types=[
        pltpu.VMEM((_L, _BSLAB), jnp.int32),
        pltpu.VMEM((_L, _DIM), jnp.float32),
        pltpu.VMEM((_NBUF, _BSLAB, _DIM), jnp.float32),
        pltpu.VMEM((_NBUF, _DT, 8 * _BSLAB), jnp.float32),
        pltpu.SemaphoreType.DMA,
        pltpu.SemaphoreType.DMA,
    ],
    compiler_params=pltpu.CompilerParams(
        use_tc_tiling_on_sc=False, needs_layout_passes=False
    ),
)
def _sc_lookup(table_hbm, xt_hbm, pos_hbm, out_hbm, idx_all, pos_v, rows_v,
               out_v, gsem, osem):
    wid = lax.axis_index("s") * _NC + lax.axis_index("c")
    b0 = wid * _BSLAB

    pltpu.sync_copy(pos_hbm, pos_v)
    pltpu.sync_copy(xt_hbm.at[:, pl.ds(b0, _BSLAB)], idx_all)

    iota16 = lax.iota(jnp.int32, 16)
    row_idx = [iota16 + 16 * k for k in range(_BSLAB // 16)]

    def fire_gather(l, b):
        pltpu.async_copy(table_hbm.at[idx_all.at[l]], rows_v.at[b], gsem)

    def wait_gather(b):
        pltpu.make_async_copy(
            table_hbm.at[idx_all.at[0]], rows_v.at[b], gsem
        ).wait()

    def fire_out(l, b):
        pltpu.async_copy(out_v.at[b], out_hbm.at[l, :, wid], osem)

    def wait_out():
        pltpu.make_async_copy(out_v.at[0], out_hbm.at[0, :, 0], osem).wait()

    def transpose_add(l, b):
        l_vec = jnp.full((16,), l, jnp.int32)

        @plsc.parallel_loop(0, _DIM, 1, unroll=2)
        def _(jj):
            rot = (iota16 + jj) & 15
            col = rot + (jj - (jj & 15))
            g = col >> 3
            inner = (col & 7) << 7
            p = plsc.load_gather(pos_v, [l_vec, col])
            for k in range(_BSLAB // 16):
                v = plsc.load_gather(rows_v.at[b], [row_idx[k], col])
                plsc.store_scatter(
                    out_v.at[b], [g, inner + row_idx[k]], v + p
                )

    def step(l, b, fire_ahead):
        if fire_ahead:
            wait_out()
            fire_gather(l + 2, (b + 2) % _NBUF)
        wait_gather(b)
        transpose_add(l, b)
        fire_out(l, b)

    pltpu.async_copy(out_v.at[2], out_hbm.at[0, :, wid], osem)
    pltpu.async_copy(out_v.at[3], out_hbm.at[1, :, wid], osem)
    fire_gather(0, 0)
    fire_gather(1, 1)

    for l in range(4):
        step(l, l % _NBUF, True)

    def group(g, carry):
        l0 = 4 + g * _NBUF
        for k in range(_NBUF):
            step(l0 + k, k, True)
        return carry

    lax.fori_loop(0, (_L - 8) // _NBUF, group, 0)

    for l in range(_L - 4, _L):
        step(l, l % _NBUF, l + 2 < _L)

    for _ in range(4):
        wait_out()


def kernel(x, table):
    pos = _pos_table()
    xt = jnp.transpose(x)
    raw = _sc_lookup(table, xt, pos)
    raw5 = raw.reshape(_L, _DT, _NW, 8, _BSLAB)
    return jnp.transpose(raw5, (2, 4, 0, 1, 3)).reshape(_B, _L, _DIM)


# speedup vs baseline: 1.7220x; 1.0836x over previous
import functools
import math

import jax
import jax.numpy as jnp
from jax import lax
from jax.experimental import pallas as pl
from jax.experimental.pallas import tpu as pltpu
from jax.experimental.pallas import tpu_sc as plsc

_VOCAB = 1000000
_DIM = 64
_B = 4096
_L = 200

_NC = 2
_NS = 16
_NW = _NC * _NS

_BSLAB = _B // _NW
_NBUF = 4
_DT = _DIM // 8


def _pos_table():
    def body(o_ref):
        i = lax.broadcasted_iota(jnp.int32, (_L, _DIM), 0).astype(jnp.float32)
        d = lax.broadcasted_iota(jnp.int32, (_L, _DIM), 1)
        d_even = ((d // 2) * 2).astype(jnp.float32)
        angle = i * jnp.exp(d_even * (-math.log(10000.0) / _DIM))
        o_ref[...] = jnp.where(d % 2 == 0, jnp.sin(angle), jnp.cos(angle))

    return pl.pallas_call(
        body, out_shape=jax.ShapeDtypeStruct((_L, _DIM), jnp.float32)
    )()


@functools.partial(
    pl.kernel,
    mesh=plsc.VectorSubcoreMesh(core_axis_name="c", subcore_axis_name="s"),
    out_type=jax.ShapeDtypeStruct((_L, _DT, _NW, 8 * _BSLAB), jnp.float32),
    scratch_types=[
        pltpu.VMEM((_L, _BSLAB), jnp.int32),
        pltpu.VMEM((_L, _DIM), jnp.float32),
        pltpu.VMEM((_NBUF, _BSLAB, _DIM), jnp.float32),
        pltpu.VMEM((_NBUF, _DT, 8 * _BSLAB), jnp.float32),
        pltpu.SemaphoreType.DMA,
        pltpu.SemaphoreType.DMA,
    ],
    compiler_params=pltpu.CompilerParams(
        use_tc_tiling_on_sc=False, needs_layout_passes=False
    ),
)
def _sc_lookup(table_hbm, xt_hbm, pos_hbm, out_hbm, idx_all, pos_v, rows_v,
               out_v, gsem, osem):
    wid = lax.axis_index("s") * _NC + lax.axis_index("c")
    b0 = wid * _BSLAB

    pltpu.sync_copy(pos_hbm, pos_v)
    pltpu.sync_copy(xt_hbm.at[:, pl.ds(b0, _BSLAB)], idx_all)

    def dbl(l, carry):
        for k in range(_BSLAB // 16):
            seg = pl.ds(k * 16, 16)
            idx_all[l, seg] = idx_all[l, seg] << 1
        return carry

    lax.fori_loop(0, _L, dbl, 0)

    iota16 = lax.iota(jnp.int32, 16)
    row_idx = [iota16 + 16 * k for k in range(_BSLAB // 16)]

    def fire_gather(l, b):
        pltpu.async_copy(table_hbm.at[idx_all.at[l]], rows_v.at[b], gsem)

    def wait_gather(b):
        pltpu.make_async_copy(
            table_hbm.at[idx_all.at[0]], rows_v.at[b], gsem
        ).wait()

    def fire_out(l, b):
        pltpu.async_copy(out_v.at[b], out_hbm.at[l, :, wid], osem)

    def wait_out():
        pltpu.make_async_copy(out_v.at[0], out_hbm.at[0, :, 0], osem).wait()

    def transpose_add(l, b):
        l_vec = jnp.full((16,), l, jnp.int32)

        @plsc.parallel_loop(0, _DIM, 1, unroll=2)
        def _(jj):
            rot = (iota16 + jj) & 15
            col = rot + (jj - (jj & 15))
            g = col >> 3
            inner = (col & 7) << 7
            p = plsc.load_gather(pos_v, [l_vec, col])
            for k in range(_BSLAB // 16):
                v = plsc.load_gather(rows_v.at[b], [row_idx[k], col])
                plsc.store_scatter(
                    out_v.at[b], [g, inner + row_idx[k]], v + p
                )

    def step(l, b, fire_ahead):
        if fire_ahead:
            wait_out()
            fire_gather(l + 2, (b + 2) % _NBUF)
        wait_gather(b)
        transpose_add(l, b)
        fire_out(l, b)

    pltpu.async_copy(out_v.at[2], out_hbm.at[0, :, wid], osem)
    pltpu.async_copy(out_v.at[3], out_hbm.at[1, :, wid], osem)
    fire_gather(0, 0)
    fire_gather(1, 1)

    for l in range(4):
        step(l, l % _NBUF, True)

    def group(g, carry):
        l0 = 4 + g * _NBUF
        for k in range(_NBUF):
            step(l0 + k, k, True)
        return carry

    lax.fori_loop(0, (_L - 8) // _NBUF, group, 0)

    for l in range(_L - 4, _L):
        step(l, l % _NBUF, l + 2 < _L)

    for _ in range(4):
        wait_out()


def kernel(x, table):
    pos = _pos_table()
    xt = jnp.transpose(x)
    tablew = jnp.pad(table, ((0, 0), (0, _DIM))).reshape(2 * _VOCAB, _DIM)
    raw = _sc_lookup(tablew, xt, pos)
    raw5 = raw.reshape(_L, _DT, _NW, 8, _BSLAB)
    return jnp.transpose(raw5, (2, 4, 0, 1, 3)).reshape(_B, _L, _DIM)


# speedup vs baseline: 2.6245x; 1.5241x over previous
import functools
import math

import jax
import jax.numpy as jnp
from jax import lax
from jax.experimental import pallas as pl
from jax.experimental.pallas import tpu as pltpu
from jax.experimental.pallas import tpu_sc as plsc

_VOCAB = 1000000
_DIM = 64
_B = 4096
_L = 200

_NC = 2
_NS = 16
_NW = _NC * _NS

_BSLAB = _B // _NW
_NBUF = 4
_DT = _DIM // 8


def _pos_table():
    def body(o_ref):
        i = lax.broadcasted_iota(jnp.int32, (_L, _DIM), 0).astype(jnp.float32)
        d = lax.broadcasted_iota(jnp.int32, (_L, _DIM), 1)
        d_even = ((d // 2) * 2).astype(jnp.float32)
        angle = i * jnp.exp(d_even * (-math.log(10000.0) / _DIM))
        o_ref[...] = jnp.where(d % 2 == 0, jnp.sin(angle), jnp.cos(angle))

    return pl.pallas_call(
        body, out_shape=jax.ShapeDtypeStruct((_L, _DIM), jnp.float32)
    )()


@functools.partial(
    pl.kernel,
    mesh=plsc.VectorSubcoreMesh(core_axis_name="c", subcore_axis_name="s"),
    out_type=jax.ShapeDtypeStruct((_L, _DT, _NW, 8 * _BSLAB), jnp.float32),
    scratch_types=[
        pltpu.VMEM((_L, _BSLAB), jnp.int32),
        pltpu.VMEM((_L, _DIM), jnp.float32),
        pltpu.VMEM((_NBUF, _BSLAB, _DIM), jnp.float32),
        pltpu.VMEM((_NBUF, _DT, 8 * _BSLAB), jnp.float32),
        pltpu.SemaphoreType.DMA,
        pltpu.SemaphoreType.DMA,
    ],
    compiler_params=pltpu.CompilerParams(
        use_tc_tiling_on_sc=False, needs_layout_passes=False
    ),
)
def _sc_lookup(table_hbm, xt_hbm, pos_hbm, out_hbm, idx_all, pos_v, rows_v,
               out_v, gsem, osem):
    wid = lax.axis_index("s") * _NC + lax.axis_index("c")
    b0 = wid * _BSLAB

    pltpu.sync_copy(pos_hbm, pos_v)
    pltpu.sync_copy(xt_hbm.at[:, pl.ds(b0, _BSLAB)], idx_all)

    def dbl(l, carry):
        for k in range(_BSLAB // 16):
            seg = pl.ds(k * 16, 16)
            idx_all[l, seg] = idx_all[l, seg] << 1
        return carry

    lax.fori_loop(0, _L, dbl, 0)

    iota16 = lax.iota(jnp.int32, 16)
    row_idx = [iota16 + 16 * k for k in range(_BSLAB // 16)]

    def fire_gather(l, b):
        pltpu.async_copy(table_hbm.at[idx_all.at[l]], rows_v.at[b], gsem)

    def wait_gather(b):
        pltpu.make_async_copy(
            table_hbm.at[idx_all.at[0]], rows_v.at[b], gsem
        ).wait()

    def fire_out(l, b):
        pltpu.async_copy(out_v.at[b], out_hbm.at[l, :, wid], osem)

    def wait_out():
        pltpu.make_async_copy(out_v.at[0], out_hbm.at[0, :, 0], osem).wait()

    def transpose_add(l, b):
        l_vec = jnp.full((16,), l, jnp.int32)

        @plsc.parallel_loop(0, _DIM, 1, unroll=2)
        def _(jj):
            rot = (iota16 + jj) & 15
            col = rot + (jj - (jj & 15))
            g = col >> 3
            inner = (col & 7) << 7
            p = plsc.load_gather(pos_v, [l_vec, col])
            for k in range(_BSLAB // 16):
                v = plsc.load_gather(rows_v.at[b], [row_idx[k], col])
                plsc.store_scatter(
                    out_v.at[b], [g, inner + row_idx[k]], v + p
                )

    def step(l, b, fire_ahead):
        if fire_ahead:
            wait_out()
            fire_gather(l + 2, (b + 2) % _NBUF)
        wait_gather(b)
        transpose_add(l, b)
        fire_out(l, b)

    pltpu.async_copy(out_v.at[2], out_hbm.at[0, :, wid], osem)
    pltpu.async_copy(out_v.at[3], out_hbm.at[1, :, wid], osem)
    fire_gather(0, 0)
    fire_gather(1, 1)

    for l in range(4):
        step(l, l % _NBUF, True)

    def group(g, carry):
        l0 = 4 + g * _NBUF
        for k in range(_NBUF):
            step(l0 + k, k, True)
        return carry

    lax.fori_loop(0, (_L - 8) // _NBUF, group, 0)

    for l in range(_L - 4, _L):
        step(l, l % _NBUF, l + 2 < _L)

    for _ in range(4):
        wait_out()


_TCHUNK = 8192


def _detile_table(tt):
    def body(x_ref, o_ref):
        xt = x_ref[...].T
        o_ref[...] = jnp.concatenate(
            [xt, jnp.zeros((_TCHUNK, _DIM), jnp.float32)], axis=1
        )

    return pl.pallas_call(
        body,
        grid=(pl.cdiv(_VOCAB, _TCHUNK),),
        in_specs=[pl.BlockSpec((_DIM, _TCHUNK), lambda j: (0, j))],
        out_specs=pl.BlockSpec((_TCHUNK, 2 * _DIM), lambda j: (j, 0)),
        out_shape=jax.ShapeDtypeStruct((_VOCAB, 2 * _DIM), jnp.float32),
    )(tt)


def kernel(x, table):
    pos = _pos_table()
    xt = jnp.transpose(x)
    tablew = _detile_table(jnp.transpose(table)).reshape(2 * _VOCAB, _DIM)
    raw = _sc_lookup(tablew, xt, pos)
    raw5 = raw.reshape(_L, _DT, _NW, 8, _BSLAB)
    return jnp.transpose(raw5, (2, 4, 0, 1, 3)).reshape(_B, _L, _DIM)


# speedup vs baseline: 2.7282x; 1.0395x over previous
import functools
import math

import jax
import jax.numpy as jnp
from jax import lax
from jax.experimental import pallas as pl
from jax.experimental.pallas import tpu as pltpu
from jax.experimental.pallas import tpu_sc as plsc

_VOCAB = 1000000
_DIM = 64
_B = 4096
_L = 200

_NC = 2
_NS = 16
_NW = _NC * _NS

_BSLAB = _B // _NW
_NBUF = 4
_DT = _DIM // 8


def _pos_table():
    def body(o_ref):
        i = lax.broadcasted_iota(jnp.int32, (_L, _DIM), 0).astype(jnp.float32)
        d = lax.broadcasted_iota(jnp.int32, (_L, _DIM), 1)
        d_even = ((d // 2) * 2).astype(jnp.float32)
        angle = i * jnp.exp(d_even * (-math.log(10000.0) / _DIM))
        o_ref[...] = jnp.where(d % 2 == 0, jnp.sin(angle), jnp.cos(angle))

    return pl.pallas_call(
        body, out_shape=jax.ShapeDtypeStruct((_L, _DIM), jnp.float32)
    )()


@functools.partial(
    pl.kernel,
    mesh=plsc.VectorSubcoreMesh(core_axis_name="c", subcore_axis_name="s"),
    out_type=jax.ShapeDtypeStruct((_L, _DT, _NW, 8 * _BSLAB), jnp.float32),
    scratch_types=[
        pltpu.VMEM((_L, _BSLAB), jnp.int32),
        pltpu.VMEM((_L, _DIM), jnp.float32),
        pltpu.VMEM((_NBUF, _BSLAB, _DIM), jnp.float32),
        pltpu.VMEM((_NBUF, _DT, 8 * _BSLAB), jnp.float32),
        pltpu.SemaphoreType.DMA,
        pltpu.SemaphoreType.DMA,
    ],
    compiler_params=pltpu.CompilerParams(
        use_tc_tiling_on_sc=False, needs_layout_passes=False
    ),
)
def _sc_lookup(table_hbm, xt_hbm, pos_hbm, out_hbm, idx_all, pos_v, rows_v,
               out_v, gsem, osem):
    wid = lax.axis_index("s") * _NC + lax.axis_index("c")
    b0 = wid * _BSLAB

    pltpu.sync_copy(pos_hbm, pos_v)
    pltpu.sync_copy(xt_hbm.at[:, pl.ds(b0, _BSLAB)], idx_all)

    def dbl(l, carry):
        for k in range(_BSLAB // 16):
            seg = pl.ds(k * 16, 16)
            idx_all[l, seg] = idx_all[l, seg] << 1
        return carry

    lax.fori_loop(0, _L, dbl, 0)

    iota16 = lax.iota(jnp.int32, 16)
    row_idx = [iota16 + 16 * k for k in range(_BSLAB // 16)]

    def fire_gather(l, b):
        pltpu.async_copy(table_hbm.at[idx_all.at[l]], rows_v.at[b], gsem)

    def wait_gather(b):
        pltpu.make_async_copy(
            table_hbm.at[idx_all.at[0]], rows_v.at[b], gsem
        ).wait()

    def fire_out(l, b):
        pltpu.async_copy(out_v.at[b], out_hbm.at[l, :, wid], osem)

    def wait_out():
        pltpu.make_async_copy(out_v.at[0], out_hbm.at[0, :, 0], osem).wait()

    def transpose_add(l, b):
        l_vec = jnp.full((16,), l, jnp.int32)

        @plsc.parallel_loop(0, _DIM, 1, unroll=2)
        def _(jj):
            rot = (iota16 + jj) & 15
            col = rot + (jj - (jj & 15))
            g = col >> 3
            inner = (col & 7) << 7
            p = plsc.load_gather(pos_v, [l_vec, col])
            for k in range(_BSLAB // 16):
                v = plsc.load_gather(rows_v.at[b], [row_idx[k], col])
                plsc.store_scatter(
                    out_v.at[b], [g, inner + row_idx[k]], v + p
                )

    def step(l, b, fire_ahead):
        if fire_ahead:
            wait_out()
            fire_gather(l + 2, (b + 2) % _NBUF)
        wait_gather(b)
        transpose_add(l, b)
        fire_out(l, b)

    pltpu.async_copy(out_v.at[2], out_hbm.at[0, :, wid], osem)
    pltpu.async_copy(out_v.at[3], out_hbm.at[1, :, wid], osem)
    fire_gather(0, 0)
    fire_gather(1, 1)

    for l in range(4):
        step(l, l % _NBUF, True)

    def group(g, carry):
        l0 = 4 + g * _NBUF
        for k in range(_NBUF):
            step(l0 + k, k, True)
        return carry

    lax.fori_loop(0, (_L - 8) // _NBUF, group, 0)

    for l in range(_L - 4, _L):
        step(l, l % _NBUF, l + 2 < _L)

    for _ in range(4):
        wait_out()


_TCHUNK = 16384


def _detile_table(tt):
    def body(x_ref, o_ref):
        xt = x_ref[...].T
        o_ref[...] = jnp.concatenate(
            [xt, jnp.zeros((_TCHUNK, _DIM), jnp.float32)], axis=1
        )

    return pl.pallas_call(
        body,
        grid=(pl.cdiv(_VOCAB, _TCHUNK),),
        in_specs=[pl.BlockSpec((_DIM, _TCHUNK), lambda j: (0, j))],
        out_specs=pl.BlockSpec((_TCHUNK, 2 * _DIM), lambda j: (j, 0)),
        out_shape=jax.ShapeDtypeStruct((_VOCAB, 2 * _DIM), jnp.float32),
    )(tt)


def kernel(x, table):
    pos = _pos_table()
    xt = jnp.transpose(x)
    tablew = _detile_table(jnp.transpose(table)).reshape(2 * _VOCAB, _DIM)
    raw = _sc_lookup(tablew, xt, pos)
    raw5 = raw.reshape(_L, _DT, _NW, 8, _BSLAB)
    return jnp.transpose(raw5, (2, 4, 0, 1, 3)).reshape(_B, _L, _DIM)


# speedup vs baseline: 2.7743x; 1.0169x over previous
import functools
import math

import jax
import jax.numpy as jnp
from jax import lax
from jax.experimental import pallas as pl
from jax.experimental.pallas import tpu as pltpu
from jax.experimental.pallas import tpu_sc as plsc

_VOCAB = 1000000
_DIM = 64
_B = 4096
_L = 200

_NC = 2
_NS = 16
_NW = _NC * _NS

_BSLAB = _B // _NW
_NBUF = 4
_DT = _DIM // 8


def _pos_table():
    def body(o_ref):
        i = lax.broadcasted_iota(jnp.int32, (_L, _DIM), 0).astype(jnp.float32)
        d = lax.broadcasted_iota(jnp.int32, (_L, _DIM), 1)
        d_even = ((d // 2) * 2).astype(jnp.float32)
        angle = i * jnp.exp(d_even * (-math.log(10000.0) / _DIM))
        o_ref[...] = jnp.where(d % 2 == 0, jnp.sin(angle), jnp.cos(angle))

    return pl.pallas_call(
        body, out_shape=jax.ShapeDtypeStruct((_L, _DIM), jnp.float32)
    )()


@functools.partial(
    pl.kernel,
    mesh=plsc.VectorSubcoreMesh(core_axis_name="c", subcore_axis_name="s"),
    out_type=jax.ShapeDtypeStruct((_L, _DT, _NW, 8 * _BSLAB), jnp.float32),
    scratch_types=[
        pltpu.VMEM((_L, _BSLAB), jnp.int32),
        pltpu.VMEM((_L, _DIM), jnp.float32),
        pltpu.VMEM((_NBUF, _BSLAB, _DIM), jnp.float32),
        pltpu.VMEM((_NBUF, _DT, 8 * _BSLAB), jnp.float32),
        pltpu.SemaphoreType.DMA,
        pltpu.SemaphoreType.DMA,
    ],
    compiler_params=pltpu.CompilerParams(
        use_tc_tiling_on_sc=False, needs_layout_passes=False
    ),
)
def _sc_lookup(table_hbm, xt_hbm, pos_hbm, out_hbm, idx_all, pos_v, rows_v,
               out_v, gsem, osem):
    wid = lax.axis_index("s") * _NC + lax.axis_index("c")
    b0 = wid * _BSLAB

    pltpu.sync_copy(pos_hbm, pos_v)
    pltpu.sync_copy(xt_hbm.at[:, pl.ds(b0, _BSLAB)], idx_all)

    def dbl(l, carry):
        for k in range(_BSLAB // 16):
            seg = pl.ds(k * 16, 16)
            idx_all[l, seg] = idx_all[l, seg] << 1
        return carry

    lax.fori_loop(0, _L, dbl, 0)

    iota16 = lax.iota(jnp.int32, 16)
    row_idx = [iota16 + 16 * k for k in range(_BSLAB // 16)]

    def fire_gather(l, b):
        pltpu.async_copy(table_hbm.at[idx_all.at[l]], rows_v.at[b], gsem)

    def wait_gather(b):
        pltpu.make_async_copy(
            table_hbm.at[idx_all.at[0]], rows_v.at[b], gsem
        ).wait()

    def fire_out(l, b):
        pltpu.async_copy(out_v.at[b], out_hbm.at[l, :, wid], osem)

    def wait_out():
        pltpu.make_async_copy(out_v.at[0], out_hbm.at[0, :, 0], osem).wait()

    def transpose_add(l, b):
        l_vec = jnp.full((16,), l, jnp.int32)

        @plsc.parallel_loop(0, _DIM, 1, unroll=2)
        def _(jj):
            rot = (iota16 + jj) & 15
            col = rot + (jj - (jj & 15))
            g = col >> 3
            inner = (col & 7) << 7
            p = plsc.load_gather(pos_v, [l_vec, col])
            for k in range(_BSLAB // 16):
                v = plsc.load_gather(rows_v.at[b], [row_idx[k], col])
                plsc.store_scatter(
                    out_v.at[b], [g, inner + row_idx[k]], v + p
                )

    def step(l, b, fire_ahead):
        if fire_ahead:
            wait_out()
            fire_gather(l + 2, (b + 2) % _NBUF)
        wait_gather(b)
        transpose_add(l, b)
        fire_out(l, b)

    pltpu.async_copy(out_v.at[2], out_hbm.at[0, :, wid], osem)
    pltpu.async_copy(out_v.at[3], out_hbm.at[1, :, wid], osem)
    fire_gather(0, 0)
    fire_gather(1, 1)

    for l in range(4):
        step(l, l % _NBUF, True)

    def group(g, carry):
        l0 = 4 + g * _NBUF
        for k in range(_NBUF):
            step(l0 + k, k, True)
        return carry

    lax.fori_loop(0, (_L - 8) // _NBUF, group, 0)

    for l in range(_L - 4, _L):
        step(l, l % _NBUF, l + 2 < _L)

    for _ in range(4):
        wait_out()


_TCHUNK = 32768


def _detile_table(tt):
    def body(x_ref, o_ref):
        xt = x_ref[...].T
        o_ref[...] = jnp.concatenate(
            [xt, jnp.zeros((_TCHUNK, _DIM), jnp.float32)], axis=1
        )

    return pl.pallas_call(
        body,
        grid=(pl.cdiv(_VOCAB, _TCHUNK),),
        in_specs=[pl.BlockSpec((_DIM, _TCHUNK), lambda j: (0, j))],
        out_specs=pl.BlockSpec((_TCHUNK, 2 * _DIM), lambda j: (j, 0)),
        out_shape=jax.ShapeDtypeStruct((_VOCAB, 2 * _DIM), jnp.float32),
    )(tt)


def kernel(x, table):
    pos = _pos_table()
    xt = jnp.transpose(x)
    tablew = _detile_table(jnp.transpose(table)).reshape(2 * _VOCAB, _DIM)
    raw = _sc_lookup(tablew, xt, pos)
    raw5 = raw.reshape(_L, _DT, _NW, 8, _BSLAB)
    return jnp.transpose(raw5, (2, 4, 0, 1, 3)).reshape(_B, _L, _DIM)
